# baseline (speedup 1.0000x reference)
import functools

import jax
import jax.numpy as jnp
from jax import lax
from jax.experimental import pallas as pl
from jax.experimental.pallas import tpu as pltpu
from jax.experimental.pallas import tpu_sc as plsc

_NC = 2
_NS = 16


def kernel(x, pe):
    batch, seq_len = x.shape
    max_len, d_model = pe.shape
    flat = seq_len * d_model
    pe_flat = pe.reshape(1, flat)
    nw = _NC * _NS
    rows_per_w = batch // nw

    mesh = plsc.VectorSubcoreMesh(core_axis_name="c", subcore_axis_name="s")

    @functools.partial(
        pl.kernel,
        mesh=mesh,
        out_type=jax.ShapeDtypeStruct((batch, flat), jnp.float32),
        scratch_types=[
            pltpu.SemaphoreType.DMA,
        ],
    )
    def sc_bcast(pe_hbm, out_hbm, sem):
        wid = lax.axis_index("s") * _NC + lax.axis_index("c")
        base = wid * rows_per_w
        for j in range(rows_per_w):
            pltpu.make_async_copy(
                pe_hbm, out_hbm.at[pl.ds(base + j, 1)], sem
            ).start()
        for j in range(rows_per_w):
            pltpu.make_async_copy(
                pe_hbm, out_hbm.at[pl.ds(base + j, 1)], sem
            ).wait()

    out = sc_bcast(pe_flat)
    return out.reshape(batch, seq_len, d_model)


# speedup vs baseline: 23.0758x; 23.0758x over previous
import functools

import jax
import jax.numpy as jnp
from jax import lax
from jax.experimental import pallas as pl
from jax.experimental.pallas import tpu as pltpu
from jax.experimental.pallas import tpu_sc as plsc

_NC = 2
_NS = 16
_R = 2
_SPA = 64


def kernel(x, pe):
    batch, seq_len = x.shape
    max_len, d_model = pe.shape
    flat = seq_len * d_model
    pe_flat = pe.reshape(1, flat)
    nw = _NC * _NS
    rows_per_w = batch // nw
    ts_rows = rows_per_w - _SPA
    n_chunks = ts_rows // _R
    fill_rows = _SPA // _NS

    mesh = plsc.VectorSubcoreMesh(core_axis_name="c", subcore_axis_name="s")

    @functools.partial(
        pl.kernel,
        mesh=mesh,
        out_type=jax.ShapeDtypeStruct((batch, flat), jnp.float32),
        scratch_types=[
            pltpu.VMEM((_R, flat), jnp.float32),
            pltpu.MemorySpace.VMEM_SHARED((_SPA, flat), jnp.float32),
            pltpu.SemaphoreType.DMA,
            pltpu.SemaphoreType.DMA,
        ],
    )
    def sc_bcast(pe_hbm, out_hbm, buf, shared, sem, sem2):
        cid = lax.axis_index("c")
        sid = lax.axis_index("s")
        wid = sid * _NC + cid
        base = wid * rows_per_w
        for r in range(_R):
            pltpu.sync_copy(pe_hbm.at[0], buf.at[r])
        for r in range(fill_rows):
            pltpu.sync_copy(pe_hbm.at[0], shared.at[sid * fill_rows + r])
        plsc.subcore_barrier()
        pltpu.make_async_copy(
            shared, out_hbm.at[pl.ds(base + ts_rows, _SPA)], sem2
        ).start()
        for j in range(n_chunks):
            pltpu.make_async_copy(
                buf, out_hbm.at[pl.ds(base + j * _R, _R)], sem
            ).start()
        for j in range(n_chunks):
            pltpu.make_async_copy(
                buf, out_hbm.at[pl.ds(base + j * _R, _R)], sem
            ).wait()
        pltpu.make_async_copy(
            shared, out_hbm.at[pl.ds(base + ts_rows, _SPA)], sem2
        ).wait()

    out = sc_bcast(pe_flat)
    return out.reshape(batch, seq_len, d_model)
